# Initial kernel scaffold; baseline (speedup 1.0000x reference)
import jax
import jax.numpy as jnp
from jax.experimental import pallas as pl


def kernel(embedding_batch, CLabel, NumofLabel, queue, queue_ptr):
    raise NotImplementedError("write your pallas kernel here")


# speedup vs baseline: 1.6527x; 1.6527x over previous
import jax
import jax.numpy as jnp
from jax.experimental import pallas as pl
from jax.experimental.pallas import tpu as pltpu

_DIM = 770
_KQ = 65536
_B = 4096
_EMB = 768
_R = 16


def _body(p_ref, emb_ref, extra_ref, q_ref, o_ref, scr_ref):
    i = pl.program_id(0)

    @pl.when(i == 0)
    def _():
        scr_ref[...] = emb_ref[...].T

    o_ref[...] = q_ref[...]
    p = pl.multiple_of(p_ref[0], _B)
    r0 = pl.multiple_of(jnp.minimum(i * _R, _EMB - _R), _R)
    emb_t = scr_ref[pl.ds(r0, _R), :]
    rows = jax.lax.broadcasted_iota(jnp.int32, (_R, 1), 0) + i * _R
    vals = jnp.where(rows < _EMB, emb_t, extra_ref[...])
    o_ref[:, pl.ds(p, _B)] = vals


def kernel(embedding_batch, CLabel, NumofLabel, queue, queue_ptr):
    n = embedding_batch.shape[0]
    idx = jnp.arange(n, dtype=jnp.float32) + (
        jnp.asarray(NumofLabel, dtype=jnp.float32) - jnp.float32(n)
    )
    extra = jnp.zeros((_R, _B), dtype=jnp.float32)
    extra = extra.at[0].set(CLabel.astype(jnp.float32))
    extra = extra.at[1].set(idx)

    ptr = queue_ptr[0]
    p = jnp.where(ptr + _B >= _KQ - 1, jnp.int32(0), ptr).astype(jnp.int32)
    p_arr = p.reshape(1)

    nblocks = pl.cdiv(_DIM, _R)
    emb_blocks = _EMB // _R

    out = pl.pallas_call(
        _body,
        grid=(nblocks,),
        in_specs=[
            pl.BlockSpec(memory_space=pltpu.SMEM),
            pl.BlockSpec((n, _EMB), lambda i: (0, 0)),
            pl.BlockSpec((_R, _B), lambda i: (0, 0)),
            pl.BlockSpec((_R, _KQ), lambda i: (i, 0)),
        ],
        out_specs=pl.BlockSpec((_R, _KQ), lambda i: (i, 0)),
        out_shape=jax.ShapeDtypeStruct((_DIM, _KQ), jnp.float32),
        scratch_shapes=[pltpu.VMEM((_EMB, _B), jnp.float32)],
        compiler_params=pltpu.CompilerParams(
            dimension_semantics=("arbitrary",),
        ),
    )(p_arr, embedding_batch, extra, queue)

    new_ptr = p + jnp.int32(_B)
    return (out, new_ptr)
